# Initial kernel scaffold; baseline (speedup 1.0000x reference)
import jax
import jax.numpy as jnp
from jax.experimental import pallas as pl


def kernel(cls_score, bbox_pred, dir_cls_pred, anchors):
    raise NotImplementedError("write your pallas kernel here")


# speedup vs baseline: 3.3531x; 3.3531x over previous
import math

import jax
import jax.numpy as jnp
from jax.experimental import pallas as pl
from jax.experimental.pallas import tpu as pltpu

_N = 107136
_ROWS = 837
_PAD_ROWS = 896
_K1 = 100
_K2 = 50
_SCORE_THR = 0.1
_NMS_THR = 0.01
_PI = math.pi
_BIG = 1 << 30


def _body(sc_ref, bp_ref, dr_ref, an_ref, out_ref, s_ref):
    f32 = jnp.float32
    l128 = jax.lax.broadcasted_iota(jnp.int32, (1, 128), 1)
    l896 = jax.lax.broadcasted_iota(jnp.int32, (1, 896), 1)
    l256 = jax.lax.broadcasted_iota(jnp.int32, (1, 256), 1)
    ri = jax.lax.broadcasted_iota(jnp.int32, (_PAD_ROWS, 128), 0)
    ci = jax.lax.broadcasted_iota(jnp.int32, (_PAD_ROWS, 128), 1)
    flat = ri * 128 + ci

    s_ref[:] = jnp.where(ri < _ROWS, jax.nn.sigmoid(sc_ref[:]), f32(-1.0))

    def _ext(vec, lanes, pos):
        return jnp.sum(jnp.where(lanes == pos, vec, f32(0.0)))

    def sel_body(i, carry):
        (xa, ya, za, wa, la, ha, ra,
         xt, yt, zt, wt, lt, ht, rt, dc, sv) = carry
        s = s_ref[:]
        gm = jnp.max(s)
        idx = jnp.min(jnp.where(s == gm, flat, _BIG))
        r = idx // 128
        l = idx % 128
        row = s_ref[pl.ds(r, 1), :]
        s_ref[pl.ds(r, 1), :] = jnp.where(l128 == l, f32(-1.0), row)
        bpr = bp_ref[pl.ds(r, 1), :]
        anr = an_ref[pl.ds(r, 1), :]
        drr = dr_ref[pl.ds(r, 1), :]
        b7 = l * 7
        av = [_ext(anr, l896, b7 + j) for j in range(7)]
        bv = [_ext(bpr, l896, b7 + j) for j in range(7)]
        d0 = _ext(drr, l256, 2 * l)
        d1 = _ext(drr, l256, 2 * l + 1)
        dcs = jnp.where(d1 > d0, f32(1.0), f32(0.0))
        put = lambda vec, val: jnp.where(l128 == i, val, vec)
        return (put(xa, av[0]), put(ya, av[1]), put(za, av[2]),
                put(wa, av[3]), put(la, av[4]), put(ha, av[5]),
                put(ra, av[6]),
                put(xt, bv[0]), put(yt, bv[1]), put(zt, bv[2]),
                put(wt, bv[3]), put(lt, bv[4]), put(ht, bv[5]),
                put(rt, bv[6]), put(dc, dcs), put(sv, gm))

    zero = jnp.zeros((1, 128), f32)
    init = tuple(zero for _ in range(16))
    (xa, ya, za, wa, la, ha, ra,
     xt, yt, zt, wt, lt, ht, rt, dc, sv) = jax.lax.fori_loop(
        0, _K1, sel_body, init)

    za2 = za + ha / 2
    diag = jnp.sqrt(la * la + wa * wa)
    xg = xt * diag + xa
    yg = yt * diag + ya
    zg = zt * ha + za2
    lg = jnp.exp(lt) * la
    wg = jnp.exp(wt) * wa
    hg = jnp.exp(ht) * ha
    rg = rt + ra
    zg = zg - hg / 2

    x1 = xg - wg / 2
    y1 = yg - lg / 2
    x2 = xg + wg / 2
    y2 = yg + lg / 2
    areas = (x2 - x1) * (y2 - y1)
    vf = jnp.where(sv > _SCORE_THR, f32(1.0), f32(0.0))

    def nms_body(i, carry):
        supf, keepf = carry
        sel = (l128 == i)
        x1i = jnp.sum(jnp.where(sel, x1, f32(0.0)))
        y1i = jnp.sum(jnp.where(sel, y1, f32(0.0)))
        x2i = jnp.sum(jnp.where(sel, x2, f32(0.0)))
        y2i = jnp.sum(jnp.where(sel, y2, f32(0.0)))
        ai = jnp.sum(jnp.where(sel, areas, f32(0.0)))
        si = jnp.sum(jnp.where(sel, supf, f32(0.0)))
        vi = jnp.sum(jnp.where(sel, vf, f32(0.0)))
        ki = jnp.logical_and(si < 0.5, vi > 0.5)
        xx1 = jnp.maximum(x1i, x1)
        yy1 = jnp.maximum(y1i, y1)
        xx2 = jnp.minimum(x2i, x2)
        yy2 = jnp.minimum(y2i, y2)
        inter = jnp.maximum(xx2 - xx1, f32(0.0)) * jnp.maximum(yy2 - yy1, f32(0.0))
        iou = inter / (ai + areas - inter + f32(1e-9))
        supn = jnp.where(
            jnp.logical_and(ki, jnp.logical_and(iou > _NMS_THR, l128 > i)),
            f32(1.0), f32(0.0))
        supf = jnp.maximum(supf, supn)
        keepf = jnp.where(jnp.logical_and(sel, ki), f32(1.0), keepf)
        return supf, keepf

    supf, keepf = jax.lax.fori_loop(0, _K1, nms_body, (zero, zero))

    ks0 = jnp.where(keepf > 0.5, sv, f32(-1.0))

    def out_body(j, carry):
        ks, o0, o1, o2, o3, o4, o5, o6, o7 = carry
        m = jnp.max(ks)
        lsel = jnp.min(jnp.where(ks == m, l128, _BIG))
        sel = (l128 == lsel)
        bx0 = jnp.sum(jnp.where(sel, xg, f32(0.0)))
        bx1 = jnp.sum(jnp.where(sel, yg, f32(0.0)))
        bx2 = jnp.sum(jnp.where(sel, zg, f32(0.0)))
        bx3 = jnp.sum(jnp.where(sel, wg, f32(0.0)))
        bx4 = jnp.sum(jnp.where(sel, lg, f32(0.0)))
        bx5 = jnp.sum(jnp.where(sel, hg, f32(0.0)))
        b6 = jnp.sum(jnp.where(sel, rg, f32(0.0)))
        dci = jnp.sum(jnp.where(sel, dc, f32(0.0)))
        dir_rot = b6 + _PI / 2 - jnp.floor(b6 + 0.5) * _PI
        b6n = dir_rot - _PI / 2 + _PI * dci
        maskf = jnp.where(m > 0.0, f32(1.0), f32(0.0))
        putj = lambda vec, val: jnp.where(l128 == j, val, vec)
        return (jnp.where(sel, f32(-1.0), ks),
                putj(o0, bx0 * maskf), putj(o1, bx1 * maskf),
                putj(o2, bx2 * maskf), putj(o3, bx3 * maskf),
                putj(o4, bx4 * maskf), putj(o5, bx5 * maskf),
                putj(o6, b6n * maskf), putj(o7, m * maskf))

    outs = jax.lax.fori_loop(0, _K2, out_body,
                             (ks0,) + tuple(zero for _ in range(8)))
    out_ref[:] = jnp.concatenate(outs[1:], axis=0)


def kernel(cls_score, bbox_pred, dir_cls_pred, anchors):
    pad = _PAD_ROWS - _ROWS
    cls_t = jnp.transpose(cls_score, (1, 2, 0)).reshape(_ROWS, 128)
    cls_t = jnp.pad(cls_t, ((0, pad), (0, 0)))
    bp_t = jnp.transpose(bbox_pred, (1, 2, 0)).reshape(_ROWS, 896)
    bp_t = jnp.pad(bp_t, ((0, pad), (0, 0)))
    dr_t = jnp.transpose(dir_cls_pred, (1, 2, 0)).reshape(_ROWS, 256)
    dr_t = jnp.pad(dr_t, ((0, pad), (0, 0)))
    an_t = jnp.pad(anchors.reshape(_ROWS, 896), ((0, pad), (0, 0)))

    res = pl.pallas_call(
        _body,
        out_shape=jax.ShapeDtypeStruct((8, 128), jnp.float32),
        scratch_shapes=[pltpu.VMEM((_PAD_ROWS, 128), jnp.float32)],
    )(cls_t, bp_t, dr_t, an_t)

    out_b = res[:7, :_K2].T
    out_s = res[7, :_K2]
    labels = jnp.where(out_s > 0.0, 0, -1).astype(jnp.int32)
    return out_b, out_s, labels
